# baseline (speedup 1.0000x reference)
import functools
import jax
import jax.numpy as jnp
from jax import lax
from jax.experimental import pallas as pl
from jax.experimental.pallas import tpu as pltpu
from jax.experimental.pallas import tpu_sc as plsc

D = 64
SCALE = 8.0
NC = 2
NS = 16
NW = NC * NS
IDX_W = 128
K = 8
CH = K * IDX_W


@functools.lru_cache(maxsize=None)
def _make_gather(B):
    per_w = B // NW
    xrows_w = per_w // IDX_W
    nch = xrows_w // K
    mesh = plsc.VectorSubcoreMesh(
        core_axis_name="c", subcore_axis_name="s", num_cores=NC, num_subcores=NS
    )

    @functools.partial(
        pl.kernel,
        out_type=jax.ShapeDtypeStruct((B, D), jnp.float32),
        mesh=mesh,
        scratch_types=[
            pltpu.VMEM((K, IDX_W), jnp.int32),
            pltpu.VMEM((CH, D), jnp.float32),
            pltpu.SemaphoreType.DMA,
        ],
        compiler_params=pltpu.CompilerParams(use_tc_tiling_on_sc=False),
    )
    def k(x_hbm, lut_hbm, out_hbm, idx_v, rows_v, gsem):
        wid = lax.axis_index("s") * NC + lax.axis_index("c")
        xbase = wid * xrows_w
        obase = wid * per_w

        @pl.loop(0, nch)
        def chunk(g):
            xoff = pl.multiple_of(xbase + g * K, 8)
            pltpu.sync_copy(x_hbm.at[pl.ds(xoff, K)], idx_v)
            cps = [
                pltpu.async_copy(
                    lut_hbm.at[idx_v.at[j]],
                    rows_v.at[pl.ds(j * IDX_W, IDX_W)],
                    gsem,
                )
                for j in range(K)
            ]
            for cp in cps:
                cp.wait()

            @pl.loop(0, CH)
            def srow(i):
                for j in range(D // 16):
                    sl = pl.ds(j * 16, 16)
                    rows_v[i, sl] = rows_v[i, sl] * SCALE

            ooff = pl.multiple_of(obase + g * CH, CH)
            pltpu.sync_copy(rows_v, out_hbm.at[pl.ds(ooff, CH)])

    return k


def kernel(x, lut):
    s0, s1 = x.shape
    B = s0 * s1
    xf = x.reshape(B // IDX_W, IDX_W).astype(jnp.int32)
    out = _make_gather(B)(xf, lut)
    return out.reshape(s0, s1, D)


# speedup vs baseline: 1.1093x; 1.1093x over previous
import functools
import jax
import jax.numpy as jnp
from jax import lax
from jax.experimental import pallas as pl
from jax.experimental.pallas import tpu as pltpu
from jax.experimental.pallas import tpu_sc as plsc

D = 64
SCALE = 8.0
NC = 2
NS = 16
NW = NC * NS
IDX_W = 128
K = 4
CH = K * IDX_W
NBUF = 3


@functools.lru_cache(maxsize=None)
def _make_gather(B):
    per_w = B // NW
    xrows_w = per_w // IDX_W
    nch = xrows_w // K
    mesh = plsc.VectorSubcoreMesh(
        core_axis_name="c", subcore_axis_name="s", num_cores=NC, num_subcores=NS
    )

    @functools.partial(
        pl.kernel,
        out_type=jax.ShapeDtypeStruct((B, D), jnp.float32),
        mesh=mesh,
        scratch_types=[
            pltpu.VMEM((xrows_w, IDX_W), jnp.int32),
            pltpu.VMEM((NBUF * CH, D), jnp.float32),
            pltpu.SemaphoreType.DMA((NBUF,)),
            pltpu.SemaphoreType.DMA((NBUF,)),
        ],
        compiler_params=pltpu.CompilerParams(use_tc_tiling_on_sc=False),
    )
    def k(x_hbm, lut_hbm, out_hbm, idx_v, rows_v, gsem, osem):
        wid = lax.axis_index("s") * NC + lax.axis_index("c")
        xbase = pl.multiple_of(wid * xrows_w, 8)
        obase = pl.multiple_of(wid * per_w, CH)

        pltpu.sync_copy(x_hbm.at[pl.ds(xbase, xrows_w)], idx_v)

        def gather_descs(g):
            slot = lax.rem(g, NBUF)
            roff = pl.multiple_of(slot * CH, CH)
            return [
                pltpu.make_async_copy(
                    lut_hbm.at[idx_v.at[g * K + j]],
                    rows_v.at[pl.ds(roff + j * IDX_W, IDX_W)],
                    gsem.at[slot],
                )
                for j in range(K)
            ]

        def out_desc(g):
            slot = lax.rem(g, NBUF)
            roff = pl.multiple_of(slot * CH, CH)
            ooff = pl.multiple_of(obase + g * CH, CH)
            return pltpu.make_async_copy(
                rows_v.at[pl.ds(roff, CH)],
                out_hbm.at[pl.ds(ooff, CH)],
                osem.at[slot],
            )

        def fire_gathers(g):
            for cp in gather_descs(g):
                cp.start()

        fire_gathers(0)

        @pl.loop(0, nch)
        def it(g):
            nxt = g + 1

            @pl.when(nxt < nch)
            def _prefetch():
                @pl.when(nxt >= NBUF)
                def _drain_old_out():
                    out_desc(nxt - NBUF).wait()

                fire_gathers(nxt)

            for cp in gather_descs(g):
                cp.wait()

            slot = lax.rem(g, NBUF)
            roff = pl.multiple_of(slot * CH, CH)

            @pl.loop(0, CH, unroll=8)
            def srow(i):
                for j in range(D // 16):
                    sl = pl.ds(j * 16, 16)
                    rows_v[roff + i, sl] = rows_v[roff + i, sl] * SCALE

            out_desc(g).start()

        for t in range(nch - NBUF, nch):
            out_desc(t).wait()

    return k


def kernel(x, lut):
    s0, s1 = x.shape
    B = s0 * s1
    xf = x.reshape(B // IDX_W, IDX_W).astype(jnp.int32)
    out = _make_gather(B)(xf, lut)
    return out.reshape(s0, s1, D)


# speedup vs baseline: 1.4748x; 1.3295x over previous
import functools
import jax
import jax.numpy as jnp
from jax import lax
from jax.experimental import pallas as pl
from jax.experimental.pallas import tpu as pltpu
from jax.experimental.pallas import tpu_sc as plsc

D = 64
SCALE = 8.0
NC = 2
NS = 16
NW = NC * NS
IDX_W = 128
K = 4
CH = K * IDX_W
NBUF = 3


@functools.lru_cache(maxsize=None)
def _make_gather(B):
    per_w = B // NW
    xrows_w = per_w // IDX_W
    nch = xrows_w // K
    mesh = plsc.VectorSubcoreMesh(
        core_axis_name="c", subcore_axis_name="s", num_cores=NC, num_subcores=NS
    )

    @functools.partial(
        pl.kernel,
        out_type=jax.ShapeDtypeStruct((B, 128), jnp.float32),
        mesh=mesh,
        scratch_types=[
            pltpu.VMEM((xrows_w, IDX_W), jnp.int32),
            pltpu.VMEM((NBUF * CH, D), jnp.float32),
            pltpu.SemaphoreType.DMA((NBUF,)),
            pltpu.SemaphoreType.DMA((NBUF,)),
        ],
        compiler_params=pltpu.CompilerParams(use_tc_tiling_on_sc=False),
    )
    def k(x_hbm, lut_hbm, out_hbm, idx_v, rows_v, gsem, osem):
        wid = lax.axis_index("s") * NC + lax.axis_index("c")
        xbase = pl.multiple_of(wid * xrows_w, 8)
        obase = pl.multiple_of(wid * per_w, CH)

        pltpu.sync_copy(x_hbm.at[pl.ds(xbase, xrows_w)], idx_v)

        def gather_descs(g):
            slot = lax.rem(g, NBUF)
            roff = pl.multiple_of(slot * CH, CH)
            return [
                pltpu.make_async_copy(
                    lut_hbm.at[idx_v.at[g * K + j]],
                    rows_v.at[pl.ds(roff + j * IDX_W, IDX_W)],
                    gsem.at[slot],
                )
                for j in range(K)
            ]

        def out_desc(g):
            slot = lax.rem(g, NBUF)
            roff = pl.multiple_of(slot * CH, CH)
            ooff = pl.multiple_of(obase + g * CH, CH)
            return pltpu.make_async_copy(
                rows_v.at[pl.ds(roff, CH)],
                out_hbm.at[pl.ds(ooff, CH), pl.ds(0, D)],
                osem.at[slot],
            )

        def fire_gathers(g):
            for cp in gather_descs(g):
                cp.start()

        fire_gathers(0)

        @pl.loop(0, nch)
        def it(g):
            nxt = g + 1

            @pl.when(nxt < nch)
            def _prefetch():
                @pl.when(nxt >= NBUF)
                def _drain_old_out():
                    out_desc(nxt - NBUF).wait()

                fire_gathers(nxt)

            for cp in gather_descs(g):
                cp.wait()

            slot = lax.rem(g, NBUF)
            roff = pl.multiple_of(slot * CH, CH)

            @pl.loop(0, CH, unroll=8)
            def srow(i):
                for j in range(D // 16):
                    sl = pl.ds(j * 16, 16)
                    rows_v[roff + i, sl] = rows_v[roff + i, sl] * SCALE

            out_desc(g).start()

        for t in range(nch - NBUF, nch):
            out_desc(t).wait()

    return k


def kernel(x, lut):
    s0, s1 = x.shape
    B = s0 * s1
    xf = x.reshape(B // IDX_W, IDX_W).astype(jnp.int32)
    out_pad = _make_gather(B)(xf, lut)
    return out_pad[:, :D].reshape(s0, s1, D)


# speedup vs baseline: 1.4749x; 1.0001x over previous
import functools
import jax
import jax.numpy as jnp
from jax import lax
from jax.experimental import pallas as pl
from jax.experimental.pallas import tpu as pltpu
from jax.experimental.pallas import tpu_sc as plsc

D = 64
SCALE = 8.0
NC = 2
NS = 16
NW = NC * NS
IDX_W = 128
K = 4
CH = K * IDX_W
NBUF = 3


@functools.lru_cache(maxsize=None)
def _make_gather(B):
    per_w = B // NW
    xrows_w = per_w // IDX_W
    nch = xrows_w // K
    mesh = plsc.VectorSubcoreMesh(
        core_axis_name="c", subcore_axis_name="s", num_cores=NC, num_subcores=NS
    )

    @functools.partial(
        pl.kernel,
        out_type=jax.ShapeDtypeStruct((B, 128), jnp.float32),
        mesh=mesh,
        scratch_types=[
            pltpu.VMEM((xrows_w, IDX_W), jnp.int32),
            pltpu.VMEM((NBUF * CH, D), jnp.float32),
            pltpu.SemaphoreType.DMA((NBUF,)),
            pltpu.SemaphoreType.DMA((NBUF,)),
        ],
        compiler_params=pltpu.CompilerParams(use_tc_tiling_on_sc=False),
    )
    def k(x_hbm, lut_hbm, out_hbm, idx_v, rows_v, gsem, osem):
        wid = lax.axis_index("s") * NC + lax.axis_index("c")
        xbase = pl.multiple_of(wid * xrows_w, 8)
        obase = pl.multiple_of(wid * per_w, CH)

        pltpu.sync_copy(x_hbm.at[pl.ds(xbase, xrows_w)], idx_v)

        def gather_descs(g):
            slot = lax.rem(g, NBUF)
            roff = pl.multiple_of(slot * CH, CH)
            return [
                pltpu.make_async_copy(
                    lut_hbm.at[idx_v.at[g * K + j]],
                    rows_v.at[pl.ds(roff + j * IDX_W, IDX_W)],
                    gsem.at[slot],
                )
                for j in range(K)
            ]

        def out_desc(g):
            slot = lax.rem(g, NBUF)
            roff = pl.multiple_of(slot * CH, CH)
            ooff = pl.multiple_of(obase + g * CH, CH)
            return pltpu.make_async_copy(
                rows_v.at[pl.ds(roff, CH)],
                out_hbm.at[pl.ds(ooff, CH), pl.ds(0, D)],
                osem.at[slot],
            )

        def fire_gathers(g):
            for cp in gather_descs(g):
                cp.start()

        def drain_gathers(g):
            slot = lax.rem(g, NBUF)
            roff = pl.multiple_of(slot * CH, CH)
            pltpu.make_async_copy(
                lut_hbm.at[pl.ds(0, CH)],
                rows_v.at[pl.ds(roff, CH)],
                gsem.at[slot],
            ).wait()

        fire_gathers(0)
        fire_gathers(1)

        @pl.loop(0, nch)
        def it(g):
            nxt = g + 2

            @pl.when(nxt < nch)
            def _prefetch():
                @pl.when(nxt >= NBUF)
                def _drain_old_out():
                    out_desc(nxt - NBUF).wait()

                fire_gathers(nxt)

            drain_gathers(g)

            slot = lax.rem(g, NBUF)
            roff = pl.multiple_of(slot * CH, CH)

            @pl.loop(0, CH, unroll=8)
            def srow(i):
                for j in range(D // 16):
                    sl = pl.ds(j * 16, 16)
                    rows_v[roff + i, sl] = rows_v[roff + i, sl] * SCALE

            out_desc(g).start()

        for t in range(nch - NBUF, nch):
            out_desc(t).wait()

    return k


def kernel(x, lut):
    s0, s1 = x.shape
    B = s0 * s1
    xf = x.reshape(B // IDX_W, IDX_W).astype(jnp.int32)
    out_pad = _make_gather(B)(xf, lut)
    return out_pad[:, :D].reshape(s0, s1, D)
